# baseline (speedup 1.0000x reference)
import functools

import numpy as np
import jax
import jax.numpy as jnp
from jax import lax
from jax.experimental import pallas as pl
from jax.experimental.pallas import tpu as pltpu
from jax.experimental.pallas import tpu_sc as plsc

_VOCAB = 100000
_DIM = 1024
_MAX_LEN = 2048
_B = 4
_S = 2048

_NC = 2
_NS = 16
_NW = _NC * _NS
_TOK = _B * _S
_TPW = _TOK // _NW
_CH = 32
_NCH = _TPW // _CH
_L = 16
_VPR = _DIM // _L


def _pe_table() -> np.ndarray:
    pos = np.arange(_MAX_LEN, dtype=np.float32)[:, None]
    i = np.arange(_DIM, dtype=np.float32)[None, :]
    angle_rates = 1.0 / np.power(10000.0, (2.0 * np.floor(i / 2.0)) / _DIM)
    angles = pos * angle_rates
    pe = np.zeros((_MAX_LEN, _DIM), dtype=np.float32)
    pe[:, 0::2] = np.sin(angles[:, 0::2])
    pe[:, 1::2] = np.cos(angles[:, 1::2])
    return pe


_PE = _pe_table()


def _embed_body(x_hbm, tab_hbm, pe_hbm, out_hbm, idx_v, buf, pe_buf, sem_g, sem_p):
    wid = lax.axis_index("s") * _NC + lax.axis_index("c")
    base = wid * _TPW
    s_base = base % _S
    pltpu.sync_copy(x_hbm.at[pl.ds(base, _TPW)], idx_v)
    for c in range(_NCH):
        off = c * _CH
        g = pltpu.async_copy(tab_hbm.at[idx_v.at[pl.ds(off, _CH)]], buf, sem_g)
        p = pltpu.async_copy(pe_hbm.at[pl.ds(s_base + off, _CH), :], pe_buf, sem_p)
        g.wait()
        p.wait()

        def _row(r, carry):
            for j in range(_VPR):
                v = pe_buf[r, pl.ds(j * _L, _L)]
                plsc.addupdate(buf.at[r, pl.ds(j * _L, _L)], v)
            return carry

        lax.fori_loop(0, _CH, _row, 0)
        pltpu.sync_copy(buf, out_hbm.at[pl.ds(base + off, _CH), :])


_embed = pl.kernel(
    _embed_body,
    out_type=jax.ShapeDtypeStruct((_TOK, _DIM), jnp.float32),
    mesh=plsc.VectorSubcoreMesh(core_axis_name="c", subcore_axis_name="s"),
    scratch_types=[
        pltpu.VMEM((_TPW,), jnp.int32),
        pltpu.VMEM((_CH, _DIM), jnp.float32),
        pltpu.VMEM((_CH, _DIM), jnp.float32),
        pltpu.SemaphoreType.DMA,
        pltpu.SemaphoreType.DMA,
    ],
)


@jax.jit
def kernel(x, token_table):
    pe = jnp.asarray(_PE)
    out = _embed(x.reshape(_TOK), token_table, pe)
    return out.reshape(_B, _S, _DIM)


# speedup vs baseline: 1.2310x; 1.2310x over previous
import numpy as np
import jax
import jax.numpy as jnp
from jax import lax
from jax.experimental import pallas as pl
from jax.experimental.pallas import tpu as pltpu
from jax.experimental.pallas import tpu_sc as plsc

_VOCAB = 100000
_DIM = 1024
_MAX_LEN = 2048
_B = 4
_S = 2048

_NC = 2
_NS = 16
_NW = _NC * _NS
_TOK = _B * _S
_PPW = _S // _NW
_CH = 16
_QPB = _PPW // _CH
_NCH = _B * _QPB
_L = 16
_VPR = _DIM // _L


def _pe_table() -> np.ndarray:
    pos = np.arange(_MAX_LEN, dtype=np.float32)[:, None]
    i = np.arange(_DIM, dtype=np.float32)[None, :]
    angle_rates = 1.0 / np.power(10000.0, (2.0 * np.floor(i / 2.0)) / _DIM)
    angles = pos * angle_rates
    pe = np.zeros((_MAX_LEN, _DIM), dtype=np.float32)
    pe[:, 0::2] = np.sin(angles[:, 0::2])
    pe[:, 1::2] = np.cos(angles[:, 1::2])
    return pe


_PE = _pe_table()


def _embed_body(x_hbm, tab_hbm, pe_hbm, out_hbm,
                idx_v, pe_buf, buf0, buf1,
                sem_pe, sem_g0, sem_g1, sem_s0, sem_s1):
    wid = lax.axis_index("s") * _NC + lax.axis_index("c")
    pbase = wid * _PPW

    pe_cp = pltpu.async_copy(pe_hbm.at[pl.ds(pbase, _PPW), :], pe_buf, sem_pe)
    for b in range(_B):
        pltpu.sync_copy(x_hbm.at[pl.ds(b * _S + pbase, _PPW)],
                        idx_v.at[pl.ds(b * _PPW, _PPW)])

    bufs = (buf0, buf1)
    gsems = (sem_g0, sem_g1)
    ssems = (sem_s0, sem_s1)

    def tok_base(c):
        b, q = c // _QPB, c % _QPB
        return b * _S + pbase + q * _CH

    def gather(c):
        ioff = (c // _QPB) * _PPW + (c % _QPB) * _CH
        return pltpu.async_copy(
            tab_hbm.at[idx_v.at[pl.ds(ioff, _CH)]], bufs[c & 1], gsems[c & 1])

    gathers = [gather(0), None]
    pe_cp.wait()
    stores = [None, None]
    for c in range(_NCH):
        p = c & 1
        gathers[p].wait()
        if c + 1 < _NCH:
            if stores[1 - p] is not None:
                stores[1 - p].wait()
            gathers[1 - p] = gather(c + 1)

        q = c % _QPB

        def _row(r, carry):
            buf = bufs[p]
            for j in range(_VPR):
                v = pe_buf[q * _CH + r, pl.ds(j * _L, _L)]
                plsc.addupdate(buf.at[r, pl.ds(j * _L, _L)], v)
            return carry

        lax.fori_loop(0, _CH, _row, 0)

        stores[p] = pltpu.async_copy(
            bufs[p], out_hbm.at[pl.ds(tok_base(c), _CH), :], ssems[p])
    for st in stores:
        st.wait()


_embed = pl.kernel(
    _embed_body,
    out_type=jax.ShapeDtypeStruct((_TOK, _DIM), jnp.float32),
    mesh=plsc.VectorSubcoreMesh(core_axis_name="c", subcore_axis_name="s"),
    scratch_types=[
        pltpu.VMEM((_B * _PPW,), jnp.int32),
        pltpu.VMEM((_PPW, _DIM), jnp.float32),
        pltpu.VMEM((_CH, _DIM), jnp.float32),
        pltpu.VMEM((_CH, _DIM), jnp.float32),
        pltpu.SemaphoreType.DMA,
        pltpu.SemaphoreType.DMA,
        pltpu.SemaphoreType.DMA,
        pltpu.SemaphoreType.DMA,
        pltpu.SemaphoreType.DMA,
    ],
)


@jax.jit
def kernel(x, token_table):
    pe = jnp.asarray(_PE)
    out = _embed(x.reshape(_TOK), token_table, pe)
    return out.reshape(_B, _S, _DIM)
